# baseline (speedup 1.0000x reference)
import functools

import jax
import jax.numpy as jnp
from jax import lax
from jax.experimental import pallas as pl
from jax.experimental.pallas import tpu as pltpu
from jax.experimental.pallas import tpu_sc as plsc

N = 10000
D = 128
K = 32
F = K * D
NW = 32
NPAD = 10240
BPW = NPAD // NW
CHUNK = 128
NODES_PER_CHUNK = CHUNK // K
NCHUNK = BPW * K // CHUNK


KB = 8


def _matmul_body(x_ref, w_ref, b_ref, y_ref):
    y = jnp.dot(x_ref[...], w_ref[...], preferred_element_type=jnp.float32)
    for kk in range(KB):
        blk = y[:, kk * D:(kk + 1) * D]
        if kk == 0:
            @pl.when(pl.program_id(0) == 0)
            def _():
                y_ref[0] = blk + b_ref[...]

            @pl.when(pl.program_id(0) != 0)
            def _():
                y_ref[0] = blk
        else:
            y_ref[kk] = blk


def _tc_matmul(x, wr, brow):
    MB = 1000
    return pl.pallas_call(
        _matmul_body,
        grid=(K // KB, N // MB),
        in_specs=[
            pl.BlockSpec((MB, D), lambda kb, i: (i, 0)),
            pl.BlockSpec((D, KB * D), lambda kb, i: (0, kb)),
            pl.BlockSpec((1, D), lambda kb, i: (0, 0)),
        ],
        out_specs=pl.BlockSpec((KB, MB, D), lambda kb, i: (kb, i, 0)),
        out_shape=jax.ShapeDtypeStruct((K, N, D), jnp.float32),
        compiler_params=pltpu.CompilerParams(
            dimension_semantics=("parallel", "parallel")
        ),
    )(x, wr, brow)


DEPTH = 6
NW1 = 16
BPW1 = NPAD // NW1
GCHUNKS = tuple((c * 64, 64) for c in range(BPW1 // 64))


def _sc_bag(y4_hbm, fidx_hbm, out_hbm, idx_v, out_v, sem):
    wid = lax.axis_index("s")
    base = wid * BPW1
    pltpu.sync_copy(fidx_hbm.at[wid], idx_v)

    def fire(k, add):
        for off, sz in GCHUNKS:
            sl = pl.ds(off, sz)
            pltpu.async_copy(
                y4_hbm.at[idx_v.at[k, sl]], out_v.at[sl], sem, add=add
            )

    def drain():
        for off, sz in GCHUNKS:
            sl = pl.ds(0, sz)
            pltpu.make_async_copy(y4_hbm.at[sl], out_v.at[sl], sem).wait()

    fire(0, False)
    drain()
    for k in range(1, 1 + DEPTH):
        fire(k, True)

    def k_body(k, carry):
        fire(k, True)
        drain()
        return carry

    lax.fori_loop(1 + DEPTH, K, k_body, 0)
    for _ in range(DEPTH):
        drain()
    pltpu.sync_copy(out_v, out_hbm.at[pl.ds(base, BPW1)])


@functools.cache
def _sc_bag_call():
    return pl.kernel(
        _sc_bag,
        out_type=jax.ShapeDtypeStruct((NPAD, D), jnp.float32),
        mesh=plsc.VectorSubcoreMesh(core_axis_name="c", subcore_axis_name="s",
                                    num_cores=1),
        scratch_types=[
            pltpu.VMEM((K, BPW1), jnp.int32),
            pltpu.VMEM((BPW1, D), jnp.float32),
            pltpu.SemaphoreType.DMA,
        ],
    )


def kernel(x, neigh_sorted_orders, W, b):
    idx = neigh_sorted_orders.astype(jnp.int32)
    wr = W.reshape(D, K, D).transpose(2, 1, 0).reshape(D, F)
    brow = b.reshape(1, D)
    y = _tc_matmul(x, wr, brow)
    y4 = y.reshape(K * N, D)
    fidx = idx + N * jnp.arange(K, dtype=jnp.int32)[None, :]
    fidx = jnp.pad(fidx, ((0, NPAD - N), (0, 0)))
    fidx = fidx.reshape(NW1, BPW1, K).transpose(0, 2, 1)
    out = _sc_bag_call()(y4, fidx)
    return out[:N]


# speedup vs baseline: 1.0008x; 1.0008x over previous
import functools

import jax
import jax.numpy as jnp
from jax import lax
from jax.experimental import pallas as pl
from jax.experimental.pallas import tpu as pltpu
from jax.experimental.pallas import tpu_sc as plsc

N = 10000
D = 128
K = 32
F = K * D
NW = 32
NPAD = 10240
BPW = NPAD // NW
CHUNK = 128
NODES_PER_CHUNK = CHUNK // K
NCHUNK = BPW * K // CHUNK


KB = 8


def _matmul_body(x_ref, w_ref, b_ref, y_ref):
    y = jnp.dot(x_ref[...], w_ref[...], preferred_element_type=jnp.float32)
    for kk in range(KB):
        blk = y[:, kk * D:(kk + 1) * D]
        if kk == 0:
            @pl.when(pl.program_id(0) == 0)
            def _():
                y_ref[0] = blk + b_ref[...]

            @pl.when(pl.program_id(0) != 0)
            def _():
                y_ref[0] = blk
        else:
            y_ref[kk] = blk


def _tc_matmul(x, wr, brow):
    MB = 1000
    return pl.pallas_call(
        _matmul_body,
        grid=(K // KB, N // MB),
        in_specs=[
            pl.BlockSpec((MB, D), lambda kb, i: (i, 0)),
            pl.BlockSpec((D, KB * D), lambda kb, i: (0, kb)),
            pl.BlockSpec((1, D), lambda kb, i: (0, 0)),
        ],
        out_specs=pl.BlockSpec((KB, MB, D), lambda kb, i: (kb, i, 0)),
        out_shape=jax.ShapeDtypeStruct((K, N, D), jnp.float32),
        compiler_params=pltpu.CompilerParams(
            dimension_semantics=("parallel", "parallel")
        ),
    )(x, wr, brow)


DEPTH = 3
NW1 = 16
BPW1 = NPAD // NW1
GCHUNKS = tuple((c * 128, 128) for c in range(BPW1 // 128))


def _sc_bag(y4_hbm, fidx_hbm, out_hbm, idx_v, out_v, sem):
    wid = lax.axis_index("s")
    base = wid * BPW1
    pltpu.sync_copy(fidx_hbm.at[wid], idx_v)

    def fire(k, add):
        for off, sz in GCHUNKS:
            sl = pl.ds(off, sz)
            pltpu.async_copy(
                y4_hbm.at[idx_v.at[k, sl]], out_v.at[sl], sem, add=add
            )

    def drain():
        for off, sz in GCHUNKS:
            sl = pl.ds(0, sz)
            pltpu.make_async_copy(y4_hbm.at[sl], out_v.at[sl], sem).wait()

    fire(0, False)
    drain()
    for k in range(1, 1 + DEPTH):
        fire(k, True)

    def k_body(k, carry):
        fire(k, True)
        drain()
        return carry

    lax.fori_loop(1 + DEPTH, K, k_body, 0)
    for _ in range(DEPTH):
        drain()
    pltpu.sync_copy(out_v, out_hbm.at[pl.ds(base, BPW1)])


@functools.cache
def _sc_bag_call():
    return pl.kernel(
        _sc_bag,
        out_type=jax.ShapeDtypeStruct((NPAD, D), jnp.float32),
        mesh=plsc.VectorSubcoreMesh(core_axis_name="c", subcore_axis_name="s",
                                    num_cores=1),
        scratch_types=[
            pltpu.VMEM((K, BPW1), jnp.int32),
            pltpu.VMEM((BPW1, D), jnp.float32),
            pltpu.SemaphoreType.DMA,
        ],
    )


def kernel(x, neigh_sorted_orders, W, b):
    idx = neigh_sorted_orders.astype(jnp.int32)
    wr = W.reshape(D, K, D).transpose(2, 1, 0).reshape(D, F)
    brow = b.reshape(1, D)
    y = _tc_matmul(x, wr, brow)
    y4 = y.reshape(K * N, D)
    fidx = idx + N * jnp.arange(K, dtype=jnp.int32)[None, :]
    fidx = jnp.pad(fidx, ((0, NPAD - N), (0, 0)))
    fidx = fidx.reshape(NW1, BPW1, K).transpose(0, 2, 1)
    out = _sc_bag_call()(y4, fidx)
    return out[:N]


# speedup vs baseline: 1.1180x; 1.1171x over previous
import functools

import jax
import jax.numpy as jnp
from jax import lax
from jax.experimental import pallas as pl
from jax.experimental.pallas import tpu as pltpu
from jax.experimental.pallas import tpu_sc as plsc

N = 10000
D = 128
K = 32
F = K * D
NW = 32
NPAD = 10240
BPW = NPAD // NW
CHUNK = 128
NODES_PER_CHUNK = CHUNK // K
NCHUNK = BPW * K // CHUNK


KB = 8


def _matmul_body(x_ref, w_ref, b_ref, y_ref):
    y = jnp.dot(x_ref[...], w_ref[...], preferred_element_type=jnp.float32)
    for kk in range(KB):
        blk = y[:, kk * D:(kk + 1) * D]
        if kk == 0:
            @pl.when(pl.program_id(0) == 0)
            def _():
                y_ref[0] = blk + b_ref[...]

            @pl.when(pl.program_id(0) != 0)
            def _():
                y_ref[0] = blk
        else:
            y_ref[kk] = blk


def _tc_matmul(x, wr, brow):
    MB = 1000
    return pl.pallas_call(
        _matmul_body,
        grid=(K // KB, N // MB),
        in_specs=[
            pl.BlockSpec((MB, D), lambda kb, i: (i, 0)),
            pl.BlockSpec((D, KB * D), lambda kb, i: (0, kb)),
            pl.BlockSpec((1, D), lambda kb, i: (0, 0)),
        ],
        out_specs=pl.BlockSpec((KB, MB, D), lambda kb, i: (kb, i, 0)),
        out_shape=jax.ShapeDtypeStruct((K, N, D), jnp.float32),
        compiler_params=pltpu.CompilerParams(
            dimension_semantics=("parallel", "parallel")
        ),
    )(x, wr, brow)


DEPTH = 6
GCHUNKS = tuple((c * 64, 64) for c in range(BPW // 64))


def _sc_bag(y4_hbm, fidx_hbm, out_hbm, idx_v, out_v, sem):
    wid = lax.axis_index("s") * 2 + lax.axis_index("c")
    base = wid * BPW
    pltpu.sync_copy(fidx_hbm.at[wid], idx_v)

    def fire(k, add):
        for off, sz in GCHUNKS:
            sl = pl.ds(off, sz)
            pltpu.async_copy(
                y4_hbm.at[idx_v.at[k, sl]], out_v.at[sl], sem, add=add
            )

    def drain():
        for off, sz in GCHUNKS:
            sl = pl.ds(0, sz)
            pltpu.make_async_copy(y4_hbm.at[sl], out_v.at[sl], sem).wait()

    fire(0, False)
    drain()
    for k in range(1, 1 + DEPTH):
        fire(k, True)

    def k_body(k, carry):
        fire(k, True)
        drain()
        return carry

    lax.fori_loop(1 + DEPTH, K, k_body, 0)
    for _ in range(DEPTH):
        drain()
    pltpu.sync_copy(out_v, out_hbm.at[pl.ds(base, BPW)])


@functools.cache
def _sc_bag_call():
    return pl.kernel(
        _sc_bag,
        out_type=jax.ShapeDtypeStruct((NPAD, D), jnp.float32),
        mesh=plsc.VectorSubcoreMesh(core_axis_name="c", subcore_axis_name="s"),
        scratch_types=[
            pltpu.VMEM((K, BPW), jnp.int32),
            pltpu.VMEM((BPW, D), jnp.float32),
            pltpu.SemaphoreType.DMA,
        ],
    )


def kernel(x, neigh_sorted_orders, W, b):
    idx = neigh_sorted_orders.astype(jnp.int32)
    wr = W.reshape(D, K, D).transpose(2, 1, 0).reshape(D, F)
    brow = b.reshape(1, D)
    y = _tc_matmul(x, wr, brow)
    y4 = y.reshape(K * N, D)
    fidx = idx + N * jnp.arange(K, dtype=jnp.int32)[None, :]
    fidx = jnp.pad(fidx, ((0, NPAD - N), (0, 0)))
    fidx = fidx.reshape(NW, BPW, K).transpose(0, 2, 1)
    out = _sc_bag_call()(y4, fidx)
    return out[:N]
